# baseline (speedup 1.0000x reference)
import functools
import math

import jax
import jax.numpy as jnp
from jax.experimental import pallas as pl
from jax.experimental.pallas import tpu as pltpu

_B, _CK, _CV, _T, _H, _W = 4, 64, 512, 16, 24, 24
_HW = _H * _W
_THW = _T * _HW
_NT = 4
_KC = _THW // _NT
_INV_SQRT_CK = 1.0 / math.sqrt(_CK)


def _decoder_kernel(mk_ref, qk_ref, mv_ref, qv_ref, out_ref,
                    m_ref, l_ref, acc_ref):
    t = pl.program_id(1)
    nt = pl.num_programs(1)

    @pl.when(t == 0)
    def _init():
        m_ref[...] = jnp.full_like(m_ref, -jnp.inf)
        l_ref[...] = jnp.zeros_like(l_ref)
        acc_ref[...] = jnp.zeros_like(acc_ref)

    mkb = mk_ref[0]
    qkb = qk_ref[0]

    ones = jnp.ones((_CK, 1), dtype=jnp.float32)
    a_sq = jax.lax.dot_general(
        mkb * mkb, ones, (((0,), (0,)), ((), ())),
        preferred_element_type=jnp.float32)

    ab = jax.lax.dot_general(
        mkb, qkb, (((0,), (0,)), ((), ())),
        preferred_element_type=jnp.float32)
    s = (2.0 * ab - a_sq) * _INV_SQRT_CK

    m_prev = m_ref[...]
    m_new = jnp.maximum(m_prev, jnp.max(s, axis=0, keepdims=True))
    p = jnp.exp(s - m_new)
    scale = jnp.exp(m_prev - m_new)
    l_ref[...] = l_ref[...] * scale + jnp.sum(p, axis=0, keepdims=True)
    m_ref[...] = m_new

    pv = jax.lax.dot_general(
        mv_ref[0], p, (((1,), (0,)), ((), ())),
        preferred_element_type=jnp.float32)
    acc_ref[...] = acc_ref[...] * scale + pv

    @pl.when(t == nt - 1)
    def _finish():
        out_ref[0, :_CV, :] = acc_ref[...] / l_ref[...]
        out_ref[0, _CV:, :] = qv_ref[0]


@functools.partial(jax.jit, static_argnames=())
def kernel(mk, qk, mv, qv):
    b = mk.shape[0]
    mk_f = mk.reshape(b, _CK, _THW)
    qk_f = qk.reshape(b, _CK, _HW)
    mv_f = mv.reshape(b, _CV, _THW)
    qv_f = qv.reshape(b, _CV, _HW)

    out = pl.pallas_call(
        _decoder_kernel,
        grid=(b, _NT),
        in_specs=[
            pl.BlockSpec((1, _CK, _KC), lambda i, t: (i, 0, t)),
            pl.BlockSpec((1, _CK, _HW), lambda i, t: (i, 0, 0)),
            pl.BlockSpec((1, _CV, _KC), lambda i, t: (i, 0, t)),
            pl.BlockSpec((1, _CV, _HW), lambda i, t: (i, 0, 0)),
        ],
        out_specs=pl.BlockSpec((1, 2 * _CV, _HW), lambda i, t: (i, 0, 0)),
        out_shape=jax.ShapeDtypeStruct((b, 2 * _CV, _HW), jnp.float32),
        scratch_shapes=[
            pltpu.VMEM((1, _HW), jnp.float32),
            pltpu.VMEM((1, _HW), jnp.float32),
            pltpu.VMEM((_CV, _HW), jnp.float32),
        ],
        compiler_params=pltpu.CompilerParams(
            dimension_semantics=("parallel", "arbitrary")),
    )(mk_f, qk_f, mv_f, qv_f)

    return out.reshape(b, 2 * _CV, _H, _W)
